# Initial kernel scaffold; baseline (speedup 1.0000x reference)
import jax
import jax.numpy as jnp
from jax.experimental import pallas as pl


def kernel(edge_index, x, W1, b1, W2, b2):
    raise NotImplementedError("write your pallas kernel here")


# speedup vs baseline: 21.6583x; 21.6583x over previous
import functools

import jax
import jax.numpy as jnp
from jax import lax
from jax.experimental import pallas as pl
from jax.experimental.pallas import tpu as pltpu
from jax.experimental.pallas import tpu_sc as plsc

N = 10000
D = 128
H = 16
C = 10
W = 16

NC = 2
NS = 16
NW = NC * NS

NT = 10240
DUMMY = N
G = 16

CH = 4096
NCHUNK = 41
E_HALF = NCHUNK * CH
E_PAD = NC * E_HALF
E_TILE = E_PAD // NW

_SC_PARAMS = None


def _zero_vec(ref, n):
    def body(i, _):
        ref[pl.ds(i * G, G)] = jnp.zeros((G,), jnp.float32)
        return 0
    lax.fori_loop(0, n // G, body, 0)


def _deg_body(dst_hbm, out_hbm, dst_v, acc_v):
    cid = lax.axis_index("c")
    sid = lax.axis_index("s")
    wid = sid * NC + cid

    _zero_vec(acc_v, NT)
    pltpu.sync_copy(dst_hbm.at[pl.ds(wid * E_TILE, E_TILE)], dst_v)

    ones = jnp.ones((G,), jnp.float32)

    def body(g, _):
        d16 = dst_v[pl.ds(g * G, G)]
        plsc.addupdate_scatter(acc_v, [d16], ones)
        return 0
    lax.fori_loop(0, E_TILE // G, body, 0)

    pltpu.sync_copy(acc_v, out_hbm.at[pl.ds(wid * NT, NT)])


def _agg_body(src_hbm, dst_hbm, yt_hbm, out_hbm,
              ycol_v, acc_v, src_v, dst_v):
    cid = lax.axis_index("c")
    sid = lax.axis_index("s")

    pltpu.sync_copy(yt_hbm.at[pl.ds(sid * NT, NT)], ycol_v)
    _zero_vec(acc_v, NT)

    base = cid * E_HALF
    for ch in range(NCHUNK):
        pltpu.sync_copy(src_hbm.at[pl.ds(base + ch * CH, CH)], src_v)
        pltpu.sync_copy(dst_hbm.at[pl.ds(base + ch * CH, CH)], dst_v)

        def body(g, _):
            s16 = src_v[pl.ds(g * G, G)]
            d16 = dst_v[pl.ds(g * G, G)]
            vals = plsc.load_gather(ycol_v, [s16])
            plsc.addupdate_scatter(acc_v, [d16], vals)
            return 0
        lax.fori_loop(0, CH // G, body, 0)

    pltpu.sync_copy(acc_v, out_hbm.at[pl.ds((cid * NS + sid) * NT, NT)])


@functools.cache
def _sc_calls():
    mesh = plsc.VectorSubcoreMesh(core_axis_name="c", subcore_axis_name="s")
    params = pltpu.CompilerParams(needs_layout_passes=False)
    deg_call = pl.kernel(
        _deg_body,
        out_type=jax.ShapeDtypeStruct((NW * NT,), jnp.float32),
        mesh=mesh,
        compiler_params=params,
        scratch_types=[
            pltpu.VMEM((E_TILE,), jnp.int32),
            pltpu.VMEM((NT,), jnp.float32),
        ],
    )
    agg_call = pl.kernel(
        _agg_body,
        out_type=jax.ShapeDtypeStruct((NW * NT,), jnp.float32),
        mesh=mesh,
        compiler_params=params,
        scratch_types=[
            pltpu.VMEM((NT,), jnp.float32),
            pltpu.VMEM((NT,), jnp.float32),
            pltpu.VMEM((CH,), jnp.int32),
            pltpu.VMEM((CH,), jnp.int32),
        ],
    )
    return deg_call, agg_call


def _tc1_body(degp_ref, x_ref, w1_ref, dinv_ref, y1t_ref):
    deg = jnp.sum(degp_ref[...], axis=0, keepdims=True)
    dinv = jnp.where(deg > 0, lax.rsqrt(jnp.maximum(deg, 1e-12)), 0.0)
    dinv_ref[...] = dinv
    xw_t = lax.dot_general(w1_ref[...], x_ref[...],
                           (((0,), (1,)), ((), ())),
                           preferred_element_type=jnp.float32)
    y1t_ref[...] = xw_t * dinv


def _tc2_body(a0_ref, a1_ref, dinv_ref, b1_ref, w2_ref, y2t_ref):
    agg_t = a0_ref[...] + a1_ref[...]
    dinv = dinv_ref[...]
    h_t = jnp.maximum(agg_t * dinv + b1_ref[...], 0.0)
    y2t_ref[...] = lax.dot_general(w2_ref[...], h_t,
                                   (((0,), (0,)), ((), ())),
                                   preferred_element_type=jnp.float32) * dinv


def _tc3_body(a0_ref, a1_ref, dinv_ref, b2_ref, out_ref):
    z = (a0_ref[...] + a1_ref[...]) * dinv_ref[...] + b2_ref[...]
    row = lax.broadcasted_iota(jnp.int32, z.shape, 0)
    valid = row < C
    zm = jnp.where(valid, z, -jnp.inf)
    m = jnp.max(zm, axis=0, keepdims=True)
    e = jnp.where(valid, jnp.exp(zm - m), 0.0)
    s = jnp.sum(e, axis=0, keepdims=True)
    out_ref[...] = z - m - jnp.log(s)


NQ = NT // 20

_tc1_call = pl.pallas_call(
    _tc1_body,
    out_shape=[
        jax.ShapeDtypeStruct((1, NT), jnp.float32),
        jax.ShapeDtypeStruct((W, NT), jnp.float32),
    ],
)

_tc2_call = pl.pallas_call(
    _tc2_body,
    out_shape=jax.ShapeDtypeStruct((W, NT), jnp.float32),
)

_tc3_call = pl.pallas_call(
    _tc3_body,
    out_shape=jax.ShapeDtypeStruct((W, NQ), jnp.float32),
)


def kernel(edge_index, x, W1, b1, W2, b2):
    e = edge_index.shape[1]
    ei = edge_index.astype(jnp.int32)
    loop = jnp.arange(N, dtype=jnp.int32)
    pad = jnp.full((E_PAD - e - N,), DUMMY, jnp.int32)
    src = jnp.concatenate([ei[0], loop, pad])
    dst = jnp.concatenate([ei[1], loop, pad])

    x_pad = jnp.pad(x.astype(jnp.float32), ((0, NT - N), (0, 0)))
    w1 = W1.astype(jnp.float32)
    b1c = b1.astype(jnp.float32).reshape(H, 1)
    w2p = jnp.pad(W2.astype(jnp.float32), ((0, 0), (0, W - C)))
    b2c = jnp.pad(b2.astype(jnp.float32), (0, W - C)).reshape(W, 1)

    deg_call, agg_call = _sc_calls()

    degp = deg_call(dst).reshape(NW, NT)
    dinv_t, y1t = _tc1_call(degp, x_pad, w1)

    agg1 = agg_call(src, dst, y1t.reshape(-1)).reshape(NC, W, NT)
    y2t = _tc2_call(agg1[0], agg1[1], dinv_t, b1c, w2p)

    agg2 = agg_call(src, dst, y2t.reshape(-1)).reshape(NC, W, NQ, 20)
    a0 = agg2[0, :, :, 0]
    a1 = agg2[1, :, :, 0]
    dv = dinv_t.reshape(1, NQ, 20)[:, :, 0]
    ls = _tc3_call(a0, a1, dv, b2c)
    return ls[:C, : (N + 19) // 20].T


# speedup vs baseline: 30.1404x; 1.3916x over previous
import functools

import jax
import jax.numpy as jnp
from jax import lax
from jax.experimental import pallas as pl
from jax.experimental.pallas import tpu as pltpu
from jax.experimental.pallas import tpu_sc as plsc

N = 10000
D = 128
H = 16
C = 10
W = 16

NC = 2
NS = 16
NW = NC * NS

NT = 10240
DUMMY = N
G = 16

CH = 4096
NCHUNK = 41
E_HALF = NCHUNK * CH
E_PAD = NC * E_HALF
E_TILE = E_PAD // NW

_SC_PARAMS = None


def _zero_vec(ref, n):
    def body(i, _):
        ref[pl.ds(i * G, G)] = jnp.zeros((G,), jnp.float32)
        return 0
    lax.fori_loop(0, n // G, body, 0)


def _deg_body(dst_hbm, out_hbm, dst_v, acc_v):
    cid = lax.axis_index("c")
    sid = lax.axis_index("s")
    wid = sid * NC + cid

    _zero_vec(acc_v, NT)
    pltpu.sync_copy(dst_hbm.at[pl.ds(wid * E_TILE, E_TILE)], dst_v)

    ones = jnp.ones((G,), jnp.float32)

    def body(g, _):
        d16 = dst_v[pl.ds(g * G, G)]
        plsc.addupdate_scatter(acc_v, [d16], ones)
        return 0
    lax.fori_loop(0, E_TILE // G, body, 0)

    pltpu.sync_copy(acc_v, out_hbm.at[pl.ds(wid * NT, NT)])


UNROLL = 4


def _agg_body(src_hbm, dst_hbm, yt_hbm, out_hbm,
              ycol_v, acc_v, src_v0, dst_v0, src_v1, dst_v1, sem0, sem1):
    cid = lax.axis_index("c")
    sid = lax.axis_index("s")

    base = cid * E_HALF
    bufs = [(src_v0, dst_v0, sem0), (src_v1, dst_v1, sem1)]

    def start(ch):
        sv, dv, sm = bufs[ch % 2]
        return (pltpu.async_copy(src_hbm.at[pl.ds(base + ch * CH, CH)], sv, sm),
                pltpu.async_copy(dst_hbm.at[pl.ds(base + ch * CH, CH)], dv, sm))

    pend = start(0)
    pltpu.sync_copy(yt_hbm.at[pl.ds(sid * NT, NT)], ycol_v)
    _zero_vec(acc_v, NT)

    for ch in range(NCHUNK):
        nxt = start(ch + 1) if ch + 1 < NCHUNK else None
        pend[0].wait()
        pend[1].wait()
        pend = nxt
        src_v, dst_v, _ = bufs[ch % 2]

        def body(gi, _):
            for u in range(UNROLL):
                off = (gi * UNROLL + u) * G
                s16 = src_v[pl.ds(off, G)]
                d16 = dst_v[pl.ds(off, G)]
                vals = plsc.load_gather(ycol_v, [s16])
                plsc.addupdate_scatter(acc_v, [d16], vals)
            return 0
        lax.fori_loop(0, CH // G // UNROLL, body, 0)

    pltpu.sync_copy(acc_v, out_hbm.at[pl.ds((cid * NS + sid) * NT, NT)])


@functools.cache
def _sc_calls():
    mesh = plsc.VectorSubcoreMesh(core_axis_name="c", subcore_axis_name="s")
    params = pltpu.CompilerParams(needs_layout_passes=False)
    deg_call = pl.kernel(
        _deg_body,
        out_type=jax.ShapeDtypeStruct((NW * NT,), jnp.float32),
        mesh=mesh,
        compiler_params=params,
        scratch_types=[
            pltpu.VMEM((E_TILE,), jnp.int32),
            pltpu.VMEM((NT,), jnp.float32),
        ],
    )
    agg_call = pl.kernel(
        _agg_body,
        out_type=jax.ShapeDtypeStruct((NW * NT,), jnp.float32),
        mesh=mesh,
        compiler_params=params,
        scratch_types=[
            pltpu.VMEM((NT,), jnp.float32),
            pltpu.VMEM((NT,), jnp.float32),
            pltpu.VMEM((CH,), jnp.int32),
            pltpu.VMEM((CH,), jnp.int32),
            pltpu.VMEM((CH,), jnp.int32),
            pltpu.VMEM((CH,), jnp.int32),
            pltpu.SemaphoreType.DMA,
            pltpu.SemaphoreType.DMA,
        ],
    )
    return deg_call, agg_call


def _tc1_body(degp_ref, x_ref, w1_ref, dinv_ref, y1t_ref):
    deg = jnp.sum(degp_ref[...], axis=0, keepdims=True)
    dinv = jnp.where(deg > 0, lax.rsqrt(jnp.maximum(deg, 1e-12)), 0.0)
    dinv_ref[...] = dinv
    xw_t = lax.dot_general(w1_ref[...], x_ref[...],
                           (((0,), (1,)), ((), ())),
                           preferred_element_type=jnp.float32)
    y1t_ref[...] = xw_t * dinv


def _tc2_body(a0_ref, a1_ref, dinv_ref, b1_ref, w2_ref, y2t_ref):
    agg_t = a0_ref[...] + a1_ref[...]
    dinv = dinv_ref[...]
    h_t = jnp.maximum(agg_t * dinv + b1_ref[...], 0.0)
    y2t_ref[...] = lax.dot_general(w2_ref[...], h_t,
                                   (((0,), (0,)), ((), ())),
                                   preferred_element_type=jnp.float32) * dinv


def _tc3_body(a0_ref, a1_ref, dinv_ref, b2_ref, out_ref):
    z = (a0_ref[...] + a1_ref[...]) * dinv_ref[...] + b2_ref[...]
    row = lax.broadcasted_iota(jnp.int32, z.shape, 0)
    valid = row < C
    zm = jnp.where(valid, z, -jnp.inf)
    m = jnp.max(zm, axis=0, keepdims=True)
    e = jnp.where(valid, jnp.exp(zm - m), 0.0)
    s = jnp.sum(e, axis=0, keepdims=True)
    out_ref[...] = z - m - jnp.log(s)


NQ = NT // 20

_tc1_call = pl.pallas_call(
    _tc1_body,
    out_shape=[
        jax.ShapeDtypeStruct((1, NT), jnp.float32),
        jax.ShapeDtypeStruct((W, NT), jnp.float32),
    ],
)

_tc2_call = pl.pallas_call(
    _tc2_body,
    out_shape=jax.ShapeDtypeStruct((W, NT), jnp.float32),
)

_tc3_call = pl.pallas_call(
    _tc3_body,
    out_shape=jax.ShapeDtypeStruct((W, NQ), jnp.float32),
)


def kernel(edge_index, x, W1, b1, W2, b2):
    e = edge_index.shape[1]
    ei = edge_index.astype(jnp.int32)
    loop = jnp.arange(N, dtype=jnp.int32)
    pad = jnp.full((E_PAD - e - N,), DUMMY, jnp.int32)
    src = jnp.concatenate([ei[0], loop, pad])
    dst = jnp.concatenate([ei[1], loop, pad])

    x_pad = jnp.pad(x.astype(jnp.float32), ((0, NT - N), (0, 0)))
    w1 = W1.astype(jnp.float32)
    b1c = b1.astype(jnp.float32).reshape(H, 1)
    w2p = jnp.pad(W2.astype(jnp.float32), ((0, 0), (0, W - C)))
    b2c = jnp.pad(b2.astype(jnp.float32), (0, W - C)).reshape(W, 1)

    deg_call, agg_call = _sc_calls()

    degp = deg_call(dst).reshape(NW, NT)
    dinv_t, y1t = _tc1_call(degp, x_pad, w1)

    agg1 = agg_call(src, dst, y1t.reshape(-1)).reshape(NC, W, NT)
    y2t = _tc2_call(agg1[0], agg1[1], dinv_t, b1c, w2p)

    agg2 = agg_call(src, dst, y2t.reshape(-1)).reshape(NC, W, NQ, 20)
    a0 = agg2[0, :, :, 0]
    a1 = agg2[1, :, :, 0]
    dv = dinv_t.reshape(1, NQ, 20)[:, :, 0]
    ls = _tc3_call(a0, a1, dv, b2c)
    return ls[:C, : (N + 19) // 20].T


# speedup vs baseline: 30.2161x; 1.0025x over previous
import functools

import jax
import jax.numpy as jnp
from jax import lax
from jax.experimental import pallas as pl
from jax.experimental.pallas import tpu as pltpu
from jax.experimental.pallas import tpu_sc as plsc

N = 10000
D = 128
H = 16
C = 10
W = 16

NC = 2
NS = 16
NW = NC * NS

NT = 10240
DUMMY = N
G = 16

CH = 4096
NCHUNK = 41
E_HALF = NCHUNK * CH
E_PAD = NC * E_HALF
E_TILE = E_PAD // NW

_SC_PARAMS = None


def _zero_vec(ref, n):
    def body(i, _):
        ref[pl.ds(i * G, G)] = jnp.zeros((G,), jnp.float32)
        return 0
    lax.fori_loop(0, n // G, body, 0)


def _deg_body(dst_hbm, out_hbm, dst_v, acc_v):
    cid = lax.axis_index("c")
    sid = lax.axis_index("s")
    wid = sid * NC + cid

    _zero_vec(acc_v, NT)
    pltpu.sync_copy(dst_hbm.at[pl.ds(wid * E_TILE, E_TILE)], dst_v)

    ones = jnp.ones((G,), jnp.float32)

    def body(g, _):
        d16 = dst_v[pl.ds(g * G, G)]
        plsc.addupdate_scatter(acc_v, [d16], ones)
        return 0
    lax.fori_loop(0, E_TILE // G, body, 0)

    pltpu.sync_copy(acc_v, out_hbm.at[pl.ds(wid * NT, NT)])


UNROLL = 8


def _agg_body(src_hbm, dst_hbm, yt_hbm, out_hbm,
              ycol_v, acc_v, src_v0, dst_v0, src_v1, dst_v1, sem0, sem1):
    cid = lax.axis_index("c")
    sid = lax.axis_index("s")

    base = cid * E_HALF
    bufs = [(src_v0, dst_v0, sem0), (src_v1, dst_v1, sem1)]

    def start(ch):
        sv, dv, sm = bufs[ch % 2]
        return (pltpu.async_copy(src_hbm.at[pl.ds(base + ch * CH, CH)], sv, sm),
                pltpu.async_copy(dst_hbm.at[pl.ds(base + ch * CH, CH)], dv, sm))

    pend = start(0)
    pltpu.sync_copy(yt_hbm.at[pl.ds(sid * NT, NT)], ycol_v)
    _zero_vec(acc_v, NT)

    for ch in range(NCHUNK):
        nxt = start(ch + 1) if ch + 1 < NCHUNK else None
        pend[0].wait()
        pend[1].wait()
        pend = nxt
        src_v, dst_v, _ = bufs[ch % 2]

        def body(gi, _):
            for u in range(UNROLL):
                off = (gi * UNROLL + u) * G
                s16 = src_v[pl.ds(off, G)]
                d16 = dst_v[pl.ds(off, G)]
                vals = plsc.load_gather(ycol_v, [s16])
                plsc.addupdate_scatter(acc_v, [d16], vals)
            return 0
        lax.fori_loop(0, CH // G // UNROLL, body, 0)

    pltpu.sync_copy(acc_v, out_hbm.at[pl.ds((cid * NS + sid) * NT, NT)])


@functools.cache
def _sc_calls():
    mesh = plsc.VectorSubcoreMesh(core_axis_name="c", subcore_axis_name="s")
    params = pltpu.CompilerParams(needs_layout_passes=False)
    deg_call = pl.kernel(
        _deg_body,
        out_type=jax.ShapeDtypeStruct((NW * NT,), jnp.float32),
        mesh=mesh,
        compiler_params=params,
        scratch_types=[
            pltpu.VMEM((E_TILE,), jnp.int32),
            pltpu.VMEM((NT,), jnp.float32),
        ],
    )
    agg_call = pl.kernel(
        _agg_body,
        out_type=jax.ShapeDtypeStruct((NW * NT,), jnp.float32),
        mesh=mesh,
        compiler_params=params,
        scratch_types=[
            pltpu.VMEM((NT,), jnp.float32),
            pltpu.VMEM((NT,), jnp.float32),
            pltpu.VMEM((CH,), jnp.int32),
            pltpu.VMEM((CH,), jnp.int32),
            pltpu.VMEM((CH,), jnp.int32),
            pltpu.VMEM((CH,), jnp.int32),
            pltpu.SemaphoreType.DMA,
            pltpu.SemaphoreType.DMA,
        ],
    )
    return deg_call, agg_call


def _tc1_body(degp_ref, x_ref, w1_ref, dinv_ref, y1t_ref):
    deg = jnp.sum(degp_ref[...], axis=0, keepdims=True)
    dinv = jnp.where(deg > 0, lax.rsqrt(jnp.maximum(deg, 1e-12)), 0.0)
    dinv_ref[...] = dinv
    xw_t = lax.dot_general(w1_ref[...], x_ref[...],
                           (((0,), (1,)), ((), ())),
                           preferred_element_type=jnp.float32)
    y1t_ref[...] = xw_t * dinv


def _tc2_body(a0_ref, a1_ref, dinv_ref, b1_ref, w2_ref, y2t_ref):
    agg_t = a0_ref[...] + a1_ref[...]
    dinv = dinv_ref[...]
    h_t = jnp.maximum(agg_t * dinv + b1_ref[...], 0.0)
    y2t_ref[...] = lax.dot_general(w2_ref[...], h_t,
                                   (((0,), (0,)), ((), ())),
                                   preferred_element_type=jnp.float32) * dinv


def _tc3_body(a0_ref, a1_ref, dinv_ref, b2_ref, out_ref):
    z = (a0_ref[...] + a1_ref[...]) * dinv_ref[...] + b2_ref[...]
    row = lax.broadcasted_iota(jnp.int32, z.shape, 0)
    valid = row < C
    zm = jnp.where(valid, z, -jnp.inf)
    m = jnp.max(zm, axis=0, keepdims=True)
    e = jnp.where(valid, jnp.exp(zm - m), 0.0)
    s = jnp.sum(e, axis=0, keepdims=True)
    out_ref[...] = z - m - jnp.log(s)


NQ = NT // 20

_tc1_call = pl.pallas_call(
    _tc1_body,
    out_shape=[
        jax.ShapeDtypeStruct((1, NT), jnp.float32),
        jax.ShapeDtypeStruct((W, NT), jnp.float32),
    ],
)

_tc2_call = pl.pallas_call(
    _tc2_body,
    out_shape=jax.ShapeDtypeStruct((W, NT), jnp.float32),
)

_tc3_call = pl.pallas_call(
    _tc3_body,
    out_shape=jax.ShapeDtypeStruct((W, NQ), jnp.float32),
)


def kernel(edge_index, x, W1, b1, W2, b2):
    e = edge_index.shape[1]
    ei = edge_index.astype(jnp.int32)
    loop = jnp.arange(N, dtype=jnp.int32)
    pad = jnp.full((E_PAD - e - N,), DUMMY, jnp.int32)
    src = jnp.concatenate([ei[0], loop, pad])
    dst = jnp.concatenate([ei[1], loop, pad])

    x_pad = jnp.pad(x.astype(jnp.float32), ((0, NT - N), (0, 0)))
    w1 = W1.astype(jnp.float32)
    b1c = b1.astype(jnp.float32).reshape(H, 1)
    w2p = jnp.pad(W2.astype(jnp.float32), ((0, 0), (0, W - C)))
    b2c = jnp.pad(b2.astype(jnp.float32), (0, W - C)).reshape(W, 1)

    deg_call, agg_call = _sc_calls()

    degp = deg_call(dst).reshape(NW, NT)
    dinv_t, y1t = _tc1_call(degp, x_pad, w1)

    agg1 = agg_call(src, dst, y1t.reshape(-1)).reshape(NC, W, NT)
    y2t = _tc2_call(agg1[0], agg1[1], dinv_t, b1c, w2p)

    agg2 = agg_call(src, dst, y2t.reshape(-1)).reshape(NC, W, NQ, 20)
    a0 = agg2[0, :, :, 0]
    a1 = agg2[1, :, :, 0]
    dv = dinv_t.reshape(1, NQ, 20)[:, :, 0]
    ls = _tc3_call(a0, a1, dv, b2c)
    return ls[:C, : (N + 19) // 20].T
